# baseline (speedup 1.0000x reference)
import jax
import jax.numpy as jnp
from jax import lax
from jax.experimental import pallas as pl
from jax.experimental.pallas import tpu as pltpu
from jax.experimental.pallas import tpu_sc as plsc

VOCAB = 1000000
MAXLEN = 200
D = 64
BATCH = 4096
SEQ = 200

NC = 2
NS = 16
NW = NC * NS

N = BATCH * SEQ
SEQ_PER_W = BATCH // NW
ROWS_PER_W = SEQ_PER_W * SEQ
NBUF = 4


def _sc_body(idx_hbm, tok_hbm, pos_hbm, out_hbm, pos_sh, idx_v, rows_v,
             g0, g1, g2, g3, w0, w1, w2, w3, p0, p1, p2, p3):
    gsem = (g0, g1, g2, g3)
    wsem = (w0, w1, w2, w3)
    psem = (p0, p1, p2, p3)
    cid = lax.axis_index("c")
    sid = lax.axis_index("s")
    wid = sid * NC + cid
    base0 = wid * ROWS_PER_W

    @pl.when(sid == 0)
    def _():
        pltpu.sync_copy(pos_hbm, pos_sh)

    pltpu.sync_copy(idx_hbm.at[pl.ds(base0, ROWS_PER_W)], idx_v)
    plsc.subcore_barrier()

    def prefill(b):
        pltpu.async_copy(pos_sh, rows_v.at[b], psem[b])

    def issue_gather(g, b):
        pltpu.make_async_copy(pos_sh, rows_v.at[b], psem[b]).wait()
        pltpu.async_copy(tok_hbm.at[idx_v.at[pl.ds(g * SEQ, SEQ)]],
                         rows_v.at[b], gsem[b], add=True)

    def wait_gather(b):
        pltpu.make_async_copy(tok_hbm.at[idx_v.at[pl.ds(0, SEQ)]],
                              rows_v.at[b], gsem[b]).wait()

    def issue_write(g, b):
        pltpu.async_copy(rows_v.at[b], out_hbm.at[pl.ds(base0 + g * SEQ, SEQ)],
                         wsem[b])

    def wait_write(b):
        pltpu.make_async_copy(rows_v.at[b], out_hbm.at[pl.ds(0, SEQ)],
                              wsem[b]).wait()

    def step(g, b):
        wait_gather(b)
        issue_write(g, b)

    for b in range(3):
        prefill(b)
    issue_gather(0, 0)
    issue_gather(1, 1)

    step(0, 0)
    prefill(3)
    issue_gather(2, 2)
    for g in range(1, NBUF):
        b = g % NBUF
        step(g, b)
        bp = (g + 3) % NBUF
        wait_write(bp)
        prefill(bp)
        issue_gather(g + 2, (g + 2) % NBUF)

    def qbody(q, c):
        for b in range(NBUF):
            g = q * NBUF + b
            step(g, b)
            bp = (b + 3) % NBUF
            wait_write(bp)
            prefill(bp)
            issue_gather(g + 2, (b + 2) % NBUF)
        return c

    lax.fori_loop(1, SEQ_PER_W // NBUF - 1, qbody, 0)

    for g in range(SEQ_PER_W - NBUF, SEQ_PER_W):
        b = g % NBUF
        step(g, b)
        if g + 3 < SEQ_PER_W:
            bp = (g + 3) % NBUF
            wait_write(bp)
            prefill(bp)
        if g + 2 < SEQ_PER_W:
            issue_gather(g + 2, (g + 2) % NBUF)

    for b in range(NBUF):
        wait_write(b)


@jax.jit
def _run(idx_flat, token_table, pos_table):
    mesh = plsc.VectorSubcoreMesh(core_axis_name="c", subcore_axis_name="s")
    f = pl.kernel(
        _sc_body,
        out_type=jax.ShapeDtypeStruct((N, D), jnp.float32),
        mesh=mesh,
        scratch_types=[
            pltpu.VMEM_SHARED((MAXLEN, D), jnp.float32),
            pltpu.VMEM((ROWS_PER_W,), jnp.int32),
            pltpu.VMEM((NBUF, SEQ, D), jnp.float32),
        ] + [pltpu.SemaphoreType.DMA] * (3 * NBUF),
        compiler_params=pltpu.CompilerParams(use_tc_tiling_on_sc=False),
    )
    return f(idx_flat, token_table, pos_table)


def kernel(inputs, token_table, pos_table):
    idx_flat = inputs.astype(jnp.int32).reshape(N)
    out = _run(idx_flat, token_table, pos_table)
    return out.reshape(BATCH, SEQ, D)
